# baseline (speedup 1.0000x reference)
import functools

import jax
import jax.numpy as jnp
from jax import lax
from jax.experimental import pallas as pl
from jax.experimental.pallas import tpu as pltpu
from jax.experimental.pallas import tpu_sc as plsc

T, D = 2048, 2048
E, K, F = 8, 2, 768

BT = 256
NBS = T * K // BT + 7
NS = NBS * BT

NC, NSUB = 2, 16
NW = NC * NSUB
TPT = T // NW
CH = 16
NCH = TPT // CH
CCH = 8
NCCH = TPT // CCH


def _router_body(x_ref, gate_ref, posa_ref, posb_ref, rwb_ref, map_ref):
    logits = lax.dot_general(
        x_ref[...].astype(jnp.bfloat16), gate_ref[...].astype(jnp.bfloat16),
        (((1,), (1,)), ((), ())), preferred_element_type=jnp.float32)
    iota = lax.broadcasted_iota(jnp.int32, (T, E), 1)
    m1 = jnp.max(logits, axis=1, keepdims=True)
    a1 = jnp.min(jnp.where(logits == m1, iota, E), axis=1, keepdims=True)
    masked = jnp.where(iota == a1, -1e30, logits)
    m2 = jnp.max(masked, axis=1, keepdims=True)
    a2 = jnp.min(jnp.where(masked == m2, iota, E), axis=1, keepdims=True)
    rw1 = jax.nn.sigmoid(m1 - m2)
    rwb_ref[...] = jnp.broadcast_to(
        jnp.concatenate([rw1, 1.0 - rw1], axis=0), (K * T, 16))

    oha = (iota == a1).astype(jnp.int32)
    ohb = (iota == a2).astype(jnp.int32)

    def _cumsum0(v):
        sh = 1
        while sh < T:
            v = v + jnp.concatenate(
                [jnp.zeros((sh, E), v.dtype), v[:T - sh]], axis=0)
            sh *= 2
        return v

    ca = _cumsum0(oha)
    cb = _cumsum0(ohb) + ca[T - 1:T, :]
    counts = cb[T - 1:T, :]
    blk = (counts + BT - 1) // BT
    ei = lax.broadcasted_iota(jnp.int32, (E, E), 0)
    ej = lax.broadcasted_iota(jnp.int32, (E, E), 1)
    ltri = (ei <= ej).astype(jnp.float32)
    blk_cum = lax.dot_general(blk.astype(jnp.float32), ltri,
                              (((1,), (0,)), ((), ())),
                              preferred_element_type=jnp.float32)
    blk_start = blk_cum.astype(jnp.int32) - blk
    base = BT * blk_start
    posa_ref[...] = jnp.sum(oha * (base + ca - 1), axis=1, keepdims=True)
    posb_ref[...] = jnp.sum(ohb * (base + cb - 1), axis=1, keepdims=True)

    biota = lax.broadcasted_iota(jnp.int32, (NBS + 1, E), 0)
    bs = jnp.broadcast_to(blk_start, (NBS + 1, E))
    mp = jnp.sum((bs <= biota).astype(jnp.int32), axis=1, keepdims=True) - 1
    nblk = jnp.sum(blk, axis=1, keepdims=True)
    sel = lax.broadcasted_iota(jnp.int32, (NBS + 1, 1), 0) == NBS
    map_ref[...] = jnp.where(sel, nblk, jnp.clip(mp, 0, E - 1))


def _router(x, gate_w):
    return pl.pallas_call(
        _router_body,
        out_shape=(
            jax.ShapeDtypeStruct((T, 1), jnp.int32),
            jax.ShapeDtypeStruct((T, 1), jnp.int32),
            jax.ShapeDtypeStruct((K * T, 16), jnp.float32),
            jax.ShapeDtypeStruct((NBS + 1, 1), jnp.int32),
        ),
    )(x, gate_w)


def _scatter_body(x_hbm, pos_hbm, xs_hbm, idx_v, rows0, rows1,
                  lsem0, lsem1, ssem0, ssem1):
    c = lax.axis_index("c")
    s = lax.axis_index("s")
    tok0 = (s * NC + c) * TPT
    prol = []
    for ch in range(NCH):
        prol.append(pltpu.async_copy(
            pos_hbm.at[pl.ds(tok0 + CH * ch, CH)], idx_v.at[ch], lsem0))
        prol.append(pltpu.async_copy(
            pos_hbm.at[pl.ds(T + tok0 + CH * ch, CH)], idx_v.at[NCH + ch],
            lsem0))
    for cp in prol:
        cp.wait()
    rows = (rows0, rows1)
    lsem = (lsem0, lsem1)
    ssem = (ssem0, ssem1)
    loads = [None] * NCH
    loads[0] = pltpu.async_copy(x_hbm.at[pl.ds(tok0, CH)], rows0, lsem0)
    stores = []
    for ch in range(NCH):
        if ch >= 1:
            sa0, sb0 = stores[ch - 1]
            sa0.wait()
            sb0.wait()
        if ch + 1 < NCH:
            loads[ch + 1] = pltpu.async_copy(
                x_hbm.at[pl.ds(tok0 + CH * (ch + 1), CH)],
                rows[(ch + 1) % 2], lsem[(ch + 1) % 2])
        loads[ch].wait()
        buf = rows[ch % 2]
        sa = pltpu.async_copy(buf, xs_hbm.at[idx_v.at[ch]], ssem[ch % 2])
        sb = pltpu.async_copy(buf, xs_hbm.at[idx_v.at[NCH + ch]],
                              ssem[ch % 2])
        stores.append((sa, sb))
    sa0, sb0 = stores[NCH - 1]
    sa0.wait()
    sb0.wait()


_scatter = functools.partial(
    pl.kernel,
    out_type=jax.ShapeDtypeStruct((NS, D), jnp.float32),
    mesh=plsc.VectorSubcoreMesh(core_axis_name="c", subcore_axis_name="s",
                                num_cores=NC, num_subcores=NSUB),
    scratch_types=[
        pltpu.VMEM((2 * NCH, CH), jnp.int32),
        pltpu.VMEM((CH, D), jnp.float32),
        pltpu.VMEM((CH, D), jnp.float32),
        pltpu.SemaphoreType.DMA,
        pltpu.SemaphoreType.DMA,
        pltpu.SemaphoreType.DMA,
        pltpu.SemaphoreType.DMA,
    ],
)(_scatter_body)


def _ffn_body(map_ref, xs_ref, wg_ref, wu_ref, wd_ref, ys_ref):
    b = pl.program_id(0)

    @pl.when(b < map_ref[NBS])
    def _compute():
        x16 = xs_ref[...].astype(jnp.bfloat16)
        wg16 = wg_ref[0].astype(jnp.bfloat16)
        wu16 = wu_ref[0].astype(jnp.bfloat16)
        g = lax.dot_general(x16, wg16, (((1,), (1,)), ((), ())),
                            preferred_element_type=jnp.float32)
        u = lax.dot_general(x16, wu16, (((1,), (1,)), ((), ())),
                            preferred_element_type=jnp.float32)
        h16 = ((g * jax.nn.sigmoid(g)) * u).astype(jnp.bfloat16)
        wd16 = wd_ref[0].astype(jnp.bfloat16)
        ys_ref[...] = lax.dot_general(h16, wd16, (((1,), (1,)), ((), ())),
                                      preferred_element_type=jnp.float32)


def _ffn(mp, xs, w_gate, w_up, w_down):
    grid_spec = pltpu.PrefetchScalarGridSpec(
        num_scalar_prefetch=1,
        grid=(NBS,),
        in_specs=[
            pl.BlockSpec((BT, D), lambda b, m: (b, 0)),
            pl.BlockSpec((1, F, D), lambda b, m: (m[b], 0, 0)),
            pl.BlockSpec((1, F, D), lambda b, m: (m[b], 0, 0)),
            pl.BlockSpec((1, D, F), lambda b, m: (m[b], 0, 0)),
        ],
        out_specs=pl.BlockSpec((BT, D), lambda b, m: (b, 0)),
    )
    return pl.pallas_call(
        _ffn_body,
        grid_spec=grid_spec,
        out_shape=jax.ShapeDtypeStruct((NS, D), jnp.float32),
    )(mp, xs, w_gate, w_up, w_down)


def _combine_body(ys_hbm, pos_hbm, rwb_hbm, out_hbm,
                  idx_v, rwb_v, bufa, bufb, obuf, gsem, osem):
    c = lax.axis_index("c")
    s = lax.axis_index("s")
    tok0 = (s * NC + c) * TPT
    prol = []
    for ch in range(NCH):
        prol.append(pltpu.async_copy(
            pos_hbm.at[pl.ds(tok0 + CH * ch, CH)], idx_v.at[ch], gsem))
        prol.append(pltpu.async_copy(
            pos_hbm.at[pl.ds(T + tok0 + CH * ch, CH)], idx_v.at[NCH + ch],
            gsem))
        prol.append(pltpu.async_copy(
            rwb_hbm.at[pl.ds(tok0 + CH * ch, CH)], rwb_v.at[ch], gsem))
        prol.append(pltpu.async_copy(
            rwb_hbm.at[pl.ds(T + tok0 + CH * ch, CH)], rwb_v.at[NCH + ch],
            gsem))
    for cp in prol:
        cp.wait()
    ocopy = None
    for ch in range(NCH):
        ga = pltpu.async_copy(ys_hbm.at[idx_v.at[ch]], bufa, gsem)
        gb = pltpu.async_copy(ys_hbm.at[idx_v.at[NCH + ch]], bufb, gsem)
        ga.wait()
        gb.wait()
        if ocopy is not None:
            ocopy.wait()

        def _add(j, carry, ch=ch):
            for i in range(CH):
                wa = rwb_v[ch, i, :]
                wb = rwb_v[NCH + ch, i, :]
                obuf[i, pl.ds(j * 16, 16)] = (
                    wa * bufa[i, pl.ds(j * 16, 16)]
                    + wb * bufb[i, pl.ds(j * 16, 16)])
            return carry

        lax.fori_loop(0, D // 16, _add, 0)
        ocopy = pltpu.async_copy(
            obuf, out_hbm.at[pl.ds(tok0 + CH * ch, CH)], osem)
    ocopy.wait()


_combine = functools.partial(
    pl.kernel,
    out_type=jax.ShapeDtypeStruct((T, D), jnp.float32),
    mesh=plsc.VectorSubcoreMesh(core_axis_name="c", subcore_axis_name="s",
                                num_cores=NC, num_subcores=NSUB),
    scratch_types=[
        pltpu.VMEM((2 * NCH, CH), jnp.int32),
        pltpu.VMEM((2 * NCH, CH, 16), jnp.float32),
        pltpu.VMEM((CH, D), jnp.float32),
        pltpu.VMEM((CH, D), jnp.float32),
        pltpu.VMEM((CH, D), jnp.float32),
        pltpu.SemaphoreType.DMA,
        pltpu.SemaphoreType.DMA,
    ],
)(_combine_body)


def kernel(hidden_states, gate_w, w_gate, w_up, w_down):
    b, s_, d_ = hidden_states.shape
    x = hidden_states.reshape(T, D)
    posa, posb, rwb, mp = _router(x, gate_w)
    pos = jnp.concatenate([posa.reshape(T), posb.reshape(T)])
    xs = _scatter(x, pos)
    ys = _ffn(mp.reshape(NBS + 1), xs, w_gate, w_up, w_down)
    out = _combine(ys, pos, rwb)
    return out.reshape(b, s_, d_)


# speedup vs baseline: 1.0954x; 1.0954x over previous
import functools

import jax
import jax.numpy as jnp
from jax import lax
from jax.experimental import pallas as pl
from jax.experimental.pallas import tpu as pltpu
from jax.experimental.pallas import tpu_sc as plsc

T, D = 2048, 2048
E, K, F = 8, 2, 768

BT = 256
NBS = T * K // BT + 7
NS = NBS * BT

NC, NSUB = 2, 16
NW = NC * NSUB
TPT = T // NW
CH = 16
NCH = TPT // CH
CCH = 8
NCCH = TPT // CCH


def _router_body(x_ref, gate_ref, posa_ref, posb_ref, rwb_ref, map_ref):
    logits = lax.dot_general(
        x_ref[...].astype(jnp.bfloat16), gate_ref[...].astype(jnp.bfloat16),
        (((1,), (1,)), ((), ())), preferred_element_type=jnp.float32)
    iota = lax.broadcasted_iota(jnp.int32, (T, E), 1)
    m1 = jnp.max(logits, axis=1, keepdims=True)
    a1 = jnp.min(jnp.where(logits == m1, iota, E), axis=1, keepdims=True)
    masked = jnp.where(iota == a1, -1e30, logits)
    m2 = jnp.max(masked, axis=1, keepdims=True)
    a2 = jnp.min(jnp.where(masked == m2, iota, E), axis=1, keepdims=True)
    rw1 = jax.nn.sigmoid(m1 - m2)
    rwb_ref[...] = jnp.broadcast_to(
        jnp.concatenate([rw1, 1.0 - rw1], axis=0), (K * T, 16))

    oha = (iota == a1).astype(jnp.int32)
    ohb = (iota == a2).astype(jnp.int32)

    def _cumsum0(v):
        sh = 1
        while sh < T:
            v = v + jnp.concatenate(
                [jnp.zeros((sh, E), v.dtype), v[:T - sh]], axis=0)
            sh *= 2
        return v

    ca = _cumsum0(oha)
    cb = _cumsum0(ohb) + ca[T - 1:T, :]
    counts = cb[T - 1:T, :]
    blk = (counts + BT - 1) // BT
    ei = lax.broadcasted_iota(jnp.int32, (E, E), 0)
    ej = lax.broadcasted_iota(jnp.int32, (E, E), 1)
    ltri = (ei <= ej).astype(jnp.float32)
    blk_cum = lax.dot_general(blk.astype(jnp.float32), ltri,
                              (((1,), (0,)), ((), ())),
                              preferred_element_type=jnp.float32)
    blk_start = blk_cum.astype(jnp.int32) - blk
    base = BT * blk_start
    posa_ref[...] = jnp.sum(oha * (base + ca - 1), axis=1, keepdims=True)
    posb_ref[...] = jnp.sum(ohb * (base + cb - 1), axis=1, keepdims=True)

    biota = lax.broadcasted_iota(jnp.int32, (NBS + 1, E), 0)
    bs = jnp.broadcast_to(blk_start, (NBS + 1, E))
    mp = jnp.sum((bs <= biota).astype(jnp.int32), axis=1, keepdims=True) - 1
    nblk = jnp.sum(blk, axis=1, keepdims=True)
    sel = lax.broadcasted_iota(jnp.int32, (NBS + 1, 1), 0) == NBS
    map_ref[...] = jnp.where(sel, nblk, jnp.clip(mp, 0, E - 1))


def _router(x, gate_w):
    return pl.pallas_call(
        _router_body,
        out_shape=(
            jax.ShapeDtypeStruct((T, 1), jnp.int32),
            jax.ShapeDtypeStruct((T, 1), jnp.int32),
            jax.ShapeDtypeStruct((K * T, 16), jnp.float32),
            jax.ShapeDtypeStruct((NBS + 1, 1), jnp.int32),
        ),
    )(x, gate_w)


def _scatter_body(x_hbm, pos_hbm, xs_hbm, idx_v, rows0, rows1,
                  lsem0, lsem1, ssem0, ssem1):
    c = lax.axis_index("c")
    s = lax.axis_index("s")
    tok0 = (s * NC + c) * TPT
    prol = []
    for ch in range(NCH):
        prol.append(pltpu.async_copy(
            pos_hbm.at[pl.ds(tok0 + CH * ch, CH)], idx_v.at[ch], lsem0))
        prol.append(pltpu.async_copy(
            pos_hbm.at[pl.ds(T + tok0 + CH * ch, CH)], idx_v.at[NCH + ch],
            lsem0))
    for cp in prol:
        cp.wait()
    rows = (rows0, rows1)
    lsem = (lsem0, lsem1)
    ssem = (ssem0, ssem1)
    loads = [None] * NCH
    loads[0] = pltpu.async_copy(x_hbm.at[pl.ds(tok0, CH)], rows0, lsem0)
    stores = []
    for ch in range(NCH):
        if ch >= 1:
            sa0, sb0 = stores[ch - 1]
            sa0.wait()
            sb0.wait()
        if ch + 1 < NCH:
            loads[ch + 1] = pltpu.async_copy(
                x_hbm.at[pl.ds(tok0 + CH * (ch + 1), CH)],
                rows[(ch + 1) % 2], lsem[(ch + 1) % 2])
        loads[ch].wait()
        buf = rows[ch % 2]
        sa = pltpu.async_copy(buf, xs_hbm.at[idx_v.at[ch]], ssem[ch % 2])
        sb = pltpu.async_copy(buf, xs_hbm.at[idx_v.at[NCH + ch]],
                              ssem[ch % 2])
        stores.append((sa, sb))
    sa0, sb0 = stores[NCH - 1]
    sa0.wait()
    sb0.wait()


_scatter = functools.partial(
    pl.kernel,
    out_type=jax.ShapeDtypeStruct((NS, D), jnp.float32),
    mesh=plsc.VectorSubcoreMesh(core_axis_name="c", subcore_axis_name="s",
                                num_cores=NC, num_subcores=NSUB),
    scratch_types=[
        pltpu.VMEM((2 * NCH, CH), jnp.int32),
        pltpu.VMEM((CH, D), jnp.float32),
        pltpu.VMEM((CH, D), jnp.float32),
        pltpu.SemaphoreType.DMA,
        pltpu.SemaphoreType.DMA,
        pltpu.SemaphoreType.DMA,
        pltpu.SemaphoreType.DMA,
    ],
)(_scatter_body)


def _ffn_body(map_ref, xs_ref, wg_ref, wu_ref, wd_ref, ys_ref):
    b = pl.program_id(0)

    @pl.when(b < map_ref[NBS])
    def _compute():
        x16 = xs_ref[...].astype(jnp.bfloat16)
        wg16 = wg_ref[0].astype(jnp.bfloat16)
        wu16 = wu_ref[0].astype(jnp.bfloat16)
        g = lax.dot_general(x16, wg16, (((1,), (1,)), ((), ())),
                            preferred_element_type=jnp.float32)
        u = lax.dot_general(x16, wu16, (((1,), (1,)), ((), ())),
                            preferred_element_type=jnp.float32)
        h16 = ((g * jax.nn.sigmoid(g)) * u).astype(jnp.bfloat16)
        wd16 = wd_ref[0].astype(jnp.bfloat16)
        ys_ref[...] = lax.dot_general(h16, wd16, (((1,), (1,)), ((), ())),
                                      preferred_element_type=jnp.float32)


def _ffn(mp, xs, w_gate, w_up, w_down):
    grid_spec = pltpu.PrefetchScalarGridSpec(
        num_scalar_prefetch=1,
        grid=(NBS,),
        in_specs=[
            pl.BlockSpec((BT, D), lambda b, m: (b, 0)),
            pl.BlockSpec((1, F, D), lambda b, m: (m[b], 0, 0)),
            pl.BlockSpec((1, F, D), lambda b, m: (m[b], 0, 0)),
            pl.BlockSpec((1, D, F), lambda b, m: (m[b], 0, 0)),
        ],
        out_specs=pl.BlockSpec((BT, D), lambda b, m: (b, 0)),
    )
    return pl.pallas_call(
        _ffn_body,
        grid_spec=grid_spec,
        out_shape=jax.ShapeDtypeStruct((NS, D), jnp.float32),
    )(mp, xs, w_gate, w_up, w_down)


def _combine_body(ys_hbm, pos_hbm, rwb_hbm, out_hbm,
                  idx_v, rwb_v, bufa, bufb, obuf, gsem, osem):
    c = lax.axis_index("c")
    s = lax.axis_index("s")
    tok0 = (s * NC + c) * TPT
    prol = []
    for ch in range(NCH):
        prol.append(pltpu.async_copy(
            pos_hbm.at[pl.ds(tok0 + CH * ch, CH)], idx_v.at[ch], gsem))
        prol.append(pltpu.async_copy(
            pos_hbm.at[pl.ds(T + tok0 + CH * ch, CH)], idx_v.at[NCH + ch],
            gsem))
        prol.append(pltpu.async_copy(
            rwb_hbm.at[pl.ds(tok0 + CH * ch, CH)], rwb_v.at[ch], gsem))
        prol.append(pltpu.async_copy(
            rwb_hbm.at[pl.ds(T + tok0 + CH * ch, CH)], rwb_v.at[NCH + ch],
            gsem))
    for cp in prol:
        cp.wait()
    ocopy = None
    for ch in range(NCH):
        ga = pltpu.async_copy(ys_hbm.at[idx_v.at[ch]], bufa, gsem)
        gb = pltpu.async_copy(ys_hbm.at[idx_v.at[NCH + ch]], bufb, gsem)
        ga.wait()
        gb.wait()
        if ocopy is not None:
            ocopy.wait()

        for i in range(CH):
            wa = rwb_v[ch, i, :]
            wb = rwb_v[NCH + ch, i, :]

            def _add(j, carry, i=i, wa=wa, wb=wb):
                obuf[i, pl.ds(j * 16, 16)] = (
                    wa * bufa[i, pl.ds(j * 16, 16)]
                    + wb * bufb[i, pl.ds(j * 16, 16)])
                return carry

            lax.fori_loop(0, D // 16, _add, 0)
        ocopy = pltpu.async_copy(
            obuf, out_hbm.at[pl.ds(tok0 + CH * ch, CH)], osem)
    ocopy.wait()


_combine = functools.partial(
    pl.kernel,
    out_type=jax.ShapeDtypeStruct((T, D), jnp.float32),
    mesh=plsc.VectorSubcoreMesh(core_axis_name="c", subcore_axis_name="s",
                                num_cores=NC, num_subcores=NSUB),
    scratch_types=[
        pltpu.VMEM((2 * NCH, CH), jnp.int32),
        pltpu.VMEM((2 * NCH, CH, 16), jnp.float32),
        pltpu.VMEM((CH, D), jnp.float32),
        pltpu.VMEM((CH, D), jnp.float32),
        pltpu.VMEM((CH, D), jnp.float32),
        pltpu.SemaphoreType.DMA,
        pltpu.SemaphoreType.DMA,
    ],
)(_combine_body)


def kernel(hidden_states, gate_w, w_gate, w_up, w_down):
    b, s_, d_ = hidden_states.shape
    x = hidden_states.reshape(T, D)
    posa, posb, rwb, mp = _router(x, gate_w)
    pos = jnp.concatenate([posa.reshape(T), posb.reshape(T)])
    xs = _scatter(x, pos)
    ys = _ffn(mp.reshape(NBS + 1), xs, w_gate, w_up, w_down)
    out = _combine(ys, pos, rwb)
    return out.reshape(b, s_, d_)
